# baseline (speedup 1.0000x reference)
import functools

import jax
import jax.numpy as jnp
from jax import lax
from jax.experimental import pallas as pl
from jax.experimental.pallas import tpu as pltpu
from jax.experimental.pallas import tpu_sc as plsc

N = 10000
E = 320000
D = 128
H = 128
C = 10
G = 64

NC = 2
NS = 16
NW = NC * NS
CHUNK = 128
N_CHUNKS = 80
GRP = 40
N_GRP = N_CHUNKS // GRP
EPW = N_CHUNKS * CHUNK
E_PAD = EPW * NW
N_PAD = 10240
RPS = N_PAD // NS


def _sc_aggregate(h, src_p, dst_p, zrows):
    mesh = plsc.VectorSubcoreMesh(core_axis_name="c", subcore_axis_name="s")

    @functools.partial(
        pl.kernel,
        mesh=mesh,
        out_type=jax.ShapeDtypeStruct((NC * N_PAD, D), jnp.float32),
        scratch_types=[
            pltpu.VMEM((GRP * CHUNK,), jnp.int32),
            pltpu.VMEM((GRP, CHUNK), jnp.int32),
            pltpu.VMEM((CHUNK, D), jnp.float32),
            pltpu.VMEM((CHUNK, D), jnp.float32),
            pltpu.VMEM_SHARED((N_PAD, D), jnp.float32),
            pltpu.SemaphoreType.DMA,
            pltpu.SemaphoreType.DMA,
            pltpu.SemaphoreType.DMA,
            pltpu.SemaphoreType.DMA,
        ],
    )
    def k(h_hbm, src_hbm, dst4_hbm, z_hbm, out_hbm, sidx, didx, rows0, rows1,
          acc, gsem0, gsem1, ssem0, ssem1):
        cid = lax.axis_index("c")
        sid = lax.axis_index("s")
        wid = sid * NC + cid

        pltpu.sync_copy(z_hbm, acc.at[pl.ds(sid * RPS, RPS)])
        plsc.subcore_barrier()

        def gather(i, rows, sem):
            return pltpu.async_copy(
                h_hbm.at[sidx.at[pl.ds(i * CHUNK, CHUNK)]], rows, sem)

        def gather_wait(i, rows, sem):
            pltpu.make_async_copy(
                h_hbm.at[sidx.at[pl.ds(i * CHUNK, CHUNK)]], rows, sem).wait()

        @pl.loop(0, N_GRP)
        def _(g):
            pltpu.sync_copy(
                src_hbm.at[pl.ds(wid * EPW + g * GRP * CHUNK, GRP * CHUNK)],
                sidx)
            pltpu.sync_copy(dst4_hbm.at[wid, g], didx)

            gather(0, rows0, gsem0)
            gather(1, rows1, gsem1)

            @pl.loop(0, GRP - 2, step=2)
            def _(i):
                gather_wait(i, rows0, gsem0)
                s0 = pltpu.async_copy(rows0, acc.at[didx.at[i]], ssem0,
                                      add=True)
                gather_wait(i + 1, rows1, gsem1)
                s1 = pltpu.async_copy(rows1, acc.at[didx.at[i + 1]], ssem1,
                                      add=True)
                s0.wait()
                gather(i + 2, rows0, gsem0)
                s1.wait()
                gather(i + 3, rows1, gsem1)

            last = GRP - 2
            gather_wait(last, rows0, gsem0)
            pltpu.sync_copy(rows0, acc.at[didx.at[last]], add=True)
            gather_wait(last + 1, rows1, gsem1)
            pltpu.sync_copy(rows1, acc.at[didx.at[last + 1]], add=True)

        plsc.subcore_barrier()
        out_row = cid * N_PAD + sid * RPS
        pltpu.sync_copy(acc.at[pl.ds(sid * RPS, RPS)],
                        out_hbm.at[pl.ds(out_row, RPS)])

    return k(h, src_p, dst_p.reshape(NW, N_GRP, GRP, CHUNK), zrows)


def _tc_layer(h, agg, Wa, ba, Wb, bb):
    def body(h_ref, a_ref, wa_ref, ba_ref, wb_ref, bb_ref, out_ref):
        s = h_ref[...] + a_ref[0, :N, :] + a_ref[1, :N, :]
        t = jnp.dot(s.astype(jnp.bfloat16), wa_ref[...].astype(jnp.bfloat16),
                    preferred_element_type=jnp.float32)
        t = jnp.maximum(t + ba_ref[...], 0.0)
        u = jnp.dot(t.astype(jnp.bfloat16), wb_ref[...].astype(jnp.bfloat16),
                    preferred_element_type=jnp.float32)
        out_ref[...] = jnp.maximum(u + bb_ref[...], 0.0)

    return pl.pallas_call(
        body,
        out_shape=jax.ShapeDtypeStruct((N, H), jnp.float32),
    )(h, agg, Wa, ba.reshape(1, H), Wb, bb.reshape(1, H))


def _tc_final(h, agg, Wa, ba, Wb, bb, batch_t, Wc, bc):
    def body(h_ref, a_ref, wa_ref, ba_ref, wb_ref, bb_ref, bt_ref, wc_ref,
             bc_ref, out_ref):
        s = h_ref[...] + a_ref[0, :N, :] + a_ref[1, :N, :]
        t = jnp.dot(s.astype(jnp.bfloat16), wa_ref[...].astype(jnp.bfloat16),
                    preferred_element_type=jnp.float32)
        t = jnp.maximum(t + ba_ref[...], 0.0)
        u = jnp.dot(t.astype(jnp.bfloat16), wb_ref[...].astype(jnp.bfloat16),
                    preferred_element_type=jnp.float32)
        h3 = jnp.maximum(u + bb_ref[...], 0.0)
        seg = (bt_ref[...] == lax.broadcasted_iota(jnp.int32, (G, 1), 0)
               ).astype(jnp.bfloat16)
        sums = jnp.dot(seg, h3.astype(jnp.bfloat16),
                       preferred_element_type=jnp.float32)
        counts = jnp.sum(seg.astype(jnp.float32), axis=1, keepdims=True)
        pooled = sums / jnp.maximum(counts, 1.0)
        out = jnp.dot(pooled, wc_ref[...], preferred_element_type=jnp.float32)
        out_ref[...] = out + bc_ref[...]

    return pl.pallas_call(
        body,
        out_shape=jax.ShapeDtypeStruct((G, C), jnp.float32),
    )(h, agg, Wa, ba.reshape(1, H), Wb, bb.reshape(1, H), batch_t, Wc,
      bc.reshape(1, C))


def kernel(x, edge_index, batch, W1_0, b1_0, W2_0, b2_0, W1_1, b1_1, W2_1,
           b2_1, W1_2, b1_2, W2_2, b2_2, Wc, bc):
    rpw = E // NW
    dpw = EPW - rpw
    dummy_src = (jnp.arange(NW * dpw, dtype=jnp.int32) * 37 % N).reshape(
        NW, dpw)
    dummy_dst = (N + jnp.arange(NW * dpw, dtype=jnp.int32) % (N_PAD - N)
                 ).reshape(NW, dpw)
    src_p = jnp.concatenate(
        [edge_index[0].reshape(NW, rpw), dummy_src], axis=1).reshape(-1)
    dst_p = jnp.concatenate(
        [edge_index[1].reshape(NW, rpw), dummy_dst], axis=1).reshape(-1)
    zrows = jnp.zeros((RPS, D), jnp.float32)
    batch_t = batch.reshape(1, N)

    agg = _sc_aggregate(x, src_p, dst_p, zrows)
    agg = agg.reshape(NC, N_PAD, D)
    h = _tc_layer(x, agg, W1_0, b1_0, W2_0, b2_0)
    agg = _sc_aggregate(h, src_p, dst_p, zrows).reshape(NC, N_PAD, D)
    h = _tc_layer(h, agg, W1_1, b1_1, W2_1, b2_1)
    agg = _sc_aggregate(h, src_p, dst_p, zrows).reshape(NC, N_PAD, D)
    return _tc_final(h, agg, W1_2, b1_2, W2_2, b2_2, batch_t, Wc, bc)


# speedup vs baseline: 1.1219x; 1.1219x over previous
import functools

import jax
import jax.numpy as jnp
from jax import lax
from jax.experimental import pallas as pl
from jax.experimental.pallas import tpu as pltpu
from jax.experimental.pallas import tpu_sc as plsc

N = 10000
E = 320000
D = 128
H = 128
C = 10
G = 64

NC = 2
NS = 16
NW = NC * NS
CHUNK = 80
N_CHUNKS = 128
GRP = 16
N_GRP = N_CHUNKS // GRP
EPW = N_CHUNKS * CHUNK
E_PAD = EPW * NW
N_PAD = 10240
RPS = N_PAD // NS


def _sc_aggregate(h, src_p, dst_p, zrows):
    mesh = plsc.VectorSubcoreMesh(core_axis_name="c", subcore_axis_name="s")

    @functools.partial(
        pl.kernel,
        mesh=mesh,
        out_type=jax.ShapeDtypeStruct((NC * N_PAD, D), jnp.float32),
        scratch_types=[
            pltpu.VMEM((GRP * CHUNK,), jnp.int32),
            pltpu.VMEM((GRP, CHUNK), jnp.int32),
            pltpu.VMEM((CHUNK, D), jnp.float32),
            pltpu.VMEM((CHUNK, D), jnp.float32),
            pltpu.VMEM((CHUNK, D), jnp.float32),
            pltpu.VMEM((CHUNK, D), jnp.float32),
            pltpu.VMEM_SHARED((N_PAD, D), jnp.float32),
            pltpu.SemaphoreType.DMA,
            pltpu.SemaphoreType.DMA,
            pltpu.SemaphoreType.DMA,
            pltpu.SemaphoreType.DMA,
            pltpu.SemaphoreType.DMA,
            pltpu.SemaphoreType.DMA,
            pltpu.SemaphoreType.DMA,
            pltpu.SemaphoreType.DMA,
        ],
    )
    def k(h_hbm, src_hbm, dst4_hbm, z_hbm, out_hbm, sidx, didx, rows0, rows1,
          rows2, rows3, acc, gsem0, gsem1, gsem2, gsem3, ssem0, ssem1, ssem2,
          ssem3):
        rows_b = [rows0, rows1, rows2, rows3]
        gsem_b = [gsem0, gsem1, gsem2, gsem3]
        ssem_b = [ssem0, ssem1, ssem2, ssem3]
        cid = lax.axis_index("c")
        sid = lax.axis_index("s")
        wid = sid * NC + cid

        pltpu.sync_copy(z_hbm, acc.at[pl.ds(sid * RPS, RPS)])
        plsc.subcore_barrier()

        def gather(i, rows, sem):
            return pltpu.async_copy(
                h_hbm.at[sidx.at[pl.ds(i * CHUNK, CHUNK)]], rows, sem)

        def gather_wait(i, rows, sem):
            pltpu.make_async_copy(
                h_hbm.at[sidx.at[pl.ds(i * CHUNK, CHUNK)]], rows, sem).wait()

        @pl.loop(0, N_GRP)
        def _(g):
            pltpu.sync_copy(
                src_hbm.at[pl.ds(wid * EPW + g * GRP * CHUNK, GRP * CHUNK)],
                sidx)
            pltpu.sync_copy(dst4_hbm.at[wid, g], didx)

            for b in range(4):
                gather(b, rows_b[b], gsem_b[b])

            @pl.loop(0, GRP - 4, step=4)
            def _(i):
                ss = []
                for b in range(4):
                    gather_wait(i + b, rows_b[b], gsem_b[b])
                    ss.append(pltpu.async_copy(
                        rows_b[b], acc.at[didx.at[i + b]], ssem_b[b],
                        add=True))
                for b in range(4):
                    ss[b].wait()
                    gather(i + b + 4, rows_b[b], gsem_b[b])

            last = GRP - 4
            for b in range(4):
                gather_wait(last + b, rows_b[b], gsem_b[b])
                pltpu.sync_copy(rows_b[b], acc.at[didx.at[last + b]],
                                add=True)

        plsc.subcore_barrier()
        out_row = cid * N_PAD + sid * RPS
        pltpu.sync_copy(acc.at[pl.ds(sid * RPS, RPS)],
                        out_hbm.at[pl.ds(out_row, RPS)])

    return k(h, src_p, dst_p.reshape(NW, N_GRP, GRP, CHUNK), zrows)


def _tc_layer(h, agg, Wa, ba, Wb, bb):
    def body(h_ref, a_ref, wa_ref, ba_ref, wb_ref, bb_ref, out_ref):
        s = h_ref[...] + a_ref[0, :N, :] + a_ref[1, :N, :]
        t = jnp.dot(s, wa_ref[...], preferred_element_type=jnp.float32)
        t = jnp.maximum(t + ba_ref[...], 0.0)
        u = jnp.dot(t, wb_ref[...], preferred_element_type=jnp.float32)
        out_ref[...] = jnp.maximum(u + bb_ref[...], 0.0)

    return pl.pallas_call(
        body,
        out_shape=jax.ShapeDtypeStruct((N, H), jnp.float32),
    )(h, agg, Wa, ba.reshape(1, H), Wb, bb.reshape(1, H))


def _tc_final(h, agg, Wa, ba, Wb, bb, batch_t, Wc, bc):
    def body(h_ref, a_ref, wa_ref, ba_ref, wb_ref, bb_ref, bt_ref, wc_ref,
             bc_ref, out_ref):
        s = h_ref[...] + a_ref[0, :N, :] + a_ref[1, :N, :]
        t = jnp.dot(s, wa_ref[...], preferred_element_type=jnp.float32)
        t = jnp.maximum(t + ba_ref[...], 0.0)
        u = jnp.dot(t, wb_ref[...], preferred_element_type=jnp.float32)
        h3 = jnp.maximum(u + bb_ref[...], 0.0)
        seg = (bt_ref[...] == lax.broadcasted_iota(jnp.int32, (G, 1), 0)
               ).astype(jnp.float32)
        sums = jnp.dot(seg, h3, preferred_element_type=jnp.float32)
        counts = jnp.sum(seg, axis=1, keepdims=True)
        pooled = sums / jnp.maximum(counts, 1.0)
        out = jnp.dot(pooled, wc_ref[...], preferred_element_type=jnp.float32)
        out_ref[...] = out + bc_ref[...]

    return pl.pallas_call(
        body,
        out_shape=jax.ShapeDtypeStruct((G, C), jnp.float32),
    )(h, agg, Wa, ba.reshape(1, H), Wb, bb.reshape(1, H), batch_t, Wc,
      bc.reshape(1, C))


def kernel(x, edge_index, batch, W1_0, b1_0, W2_0, b2_0, W1_1, b1_1, W2_1,
           b2_1, W1_2, b1_2, W2_2, b2_2, Wc, bc):
    rpw = E // NW
    dpw = EPW - rpw
    dummy_src = (jnp.arange(NW * dpw, dtype=jnp.int32) * 37 % N).reshape(
        NW, dpw)
    dummy_dst = (N + jnp.arange(NW * dpw, dtype=jnp.int32) % (N_PAD - N)
                 ).reshape(NW, dpw)
    src_p = jnp.concatenate(
        [edge_index[0].reshape(NW, rpw), dummy_src], axis=1).reshape(-1)
    dst_p = jnp.concatenate(
        [edge_index[1].reshape(NW, rpw), dummy_dst], axis=1).reshape(-1)
    zrows = jnp.zeros((RPS, D), jnp.float32)
    batch_t = batch.reshape(1, N)

    agg = _sc_aggregate(x, src_p, dst_p, zrows)
    agg = agg.reshape(NC, N_PAD, D)
    h = _tc_layer(x, agg, W1_0, b1_0, W2_0, b2_0)
    agg = _sc_aggregate(h, src_p, dst_p, zrows).reshape(NC, N_PAD, D)
    h = _tc_layer(h, agg, W1_1, b1_1, W2_1, b2_1)
    agg = _sc_aggregate(h, src_p, dst_p, zrows).reshape(NC, N_PAD, D)
    return _tc_final(h, agg, W1_2, b1_2, W2_2, b2_2, batch_t, Wc, bc)


# speedup vs baseline: 1.1986x; 1.0684x over previous
import functools

import jax
import jax.numpy as jnp
from jax import lax
from jax.experimental import pallas as pl
from jax.experimental.pallas import tpu as pltpu
from jax.experimental.pallas import tpu_sc as plsc

N = 10000
E = 320000
D = 128
H = 128
C = 10
G = 64

NC = 2
NS = 16
NW = NC * NS
CHUNK = 80
N_CHUNKS = 128
GRP = 32
N_GRP = N_CHUNKS // GRP
EPW = N_CHUNKS * CHUNK
E_PAD = EPW * NW
N_PAD = 10240
RPS = N_PAD // NS


def _sc_aggregate(h, src_p, dst_p, zrows):
    mesh = plsc.VectorSubcoreMesh(core_axis_name="c", subcore_axis_name="s")

    @functools.partial(
        pl.kernel,
        mesh=mesh,
        out_type=jax.ShapeDtypeStruct((NC * N_PAD, D), jnp.float32),
        scratch_types=[
            pltpu.VMEM((GRP * CHUNK,), jnp.int32),
            pltpu.VMEM((GRP, CHUNK), jnp.int32),
            pltpu.VMEM((CHUNK, D), jnp.float32),
            pltpu.VMEM((CHUNK, D), jnp.float32),
            pltpu.VMEM((CHUNK, D), jnp.float32),
            pltpu.VMEM((CHUNK, D), jnp.float32),
            pltpu.VMEM_SHARED((N_PAD, D), jnp.float32),
            pltpu.SemaphoreType.DMA,
            pltpu.SemaphoreType.DMA,
            pltpu.SemaphoreType.DMA,
            pltpu.SemaphoreType.DMA,
            pltpu.SemaphoreType.DMA,
            pltpu.SemaphoreType.DMA,
            pltpu.SemaphoreType.DMA,
            pltpu.SemaphoreType.DMA,
        ],
    )
    def k(h_hbm, src_hbm, dst4_hbm, z_hbm, out_hbm, sidx, didx, rows0, rows1,
          rows2, rows3, acc, gsem0, gsem1, gsem2, gsem3, ssem0, ssem1, ssem2,
          ssem3):
        rows_b = [rows0, rows1, rows2, rows3]
        gsem_b = [gsem0, gsem1, gsem2, gsem3]
        ssem_b = [ssem0, ssem1, ssem2, ssem3]
        cid = lax.axis_index("c")
        sid = lax.axis_index("s")
        wid = sid * NC + cid

        pltpu.sync_copy(z_hbm, acc.at[pl.ds(sid * RPS, RPS)])
        plsc.subcore_barrier()

        def gather(i, rows, sem):
            return pltpu.async_copy(
                h_hbm.at[sidx.at[pl.ds(i * CHUNK, CHUNK)]], rows, sem)

        def gather_wait(i, rows, sem):
            pltpu.make_async_copy(
                h_hbm.at[sidx.at[pl.ds(i * CHUNK, CHUNK)]], rows, sem).wait()

        @pl.loop(0, N_GRP)
        def _(g):
            pltpu.sync_copy(
                src_hbm.at[pl.ds(wid * EPW + g * GRP * CHUNK, GRP * CHUNK)],
                sidx)
            pltpu.sync_copy(dst4_hbm.at[wid, g], didx)

            for b in range(4):
                gather(b, rows_b[b], gsem_b[b])

            @pl.loop(0, GRP - 4, step=4)
            def _(i):
                ss = []
                for b in range(4):
                    gather_wait(i + b, rows_b[b], gsem_b[b])
                    ss.append(pltpu.async_copy(
                        rows_b[b], acc.at[didx.at[i + b]], ssem_b[b],
                        add=True))
                for b in range(4):
                    ss[b].wait()
                    gather(i + b + 4, rows_b[b], gsem_b[b])

            last = GRP - 4
            for b in range(4):
                gather_wait(last + b, rows_b[b], gsem_b[b])
                pltpu.sync_copy(rows_b[b], acc.at[didx.at[last + b]],
                                add=True)

        plsc.subcore_barrier()
        out_row = cid * N_PAD + sid * RPS
        pltpu.sync_copy(acc.at[pl.ds(sid * RPS, RPS)],
                        out_hbm.at[pl.ds(out_row, RPS)])

    return k(h, src_p, dst_p.reshape(NW, N_GRP, GRP, CHUNK), zrows)


def _tc_layer(h, agg, Wa, ba, Wb, bb):
    def body(h_ref, a_ref, wa_ref, ba_ref, wb_ref, bb_ref, out_ref):
        s = h_ref[...] + a_ref[0, :N, :] + a_ref[1, :N, :]
        t = jnp.dot(s, wa_ref[...], preferred_element_type=jnp.float32)
        t = jnp.maximum(t + ba_ref[...], 0.0)
        u = jnp.dot(t, wb_ref[...], preferred_element_type=jnp.float32)
        out_ref[...] = jnp.maximum(u + bb_ref[...], 0.0)

    return pl.pallas_call(
        body,
        out_shape=jax.ShapeDtypeStruct((N, H), jnp.float32),
    )(h, agg, Wa, ba.reshape(1, H), Wb, bb.reshape(1, H))


def _tc_final(h, agg, Wa, ba, Wb, bb, batch_t, Wc, bc):
    def body(h_ref, a_ref, wa_ref, ba_ref, wb_ref, bb_ref, bt_ref, wc_ref,
             bc_ref, out_ref):
        s = h_ref[...] + a_ref[0, :N, :] + a_ref[1, :N, :]
        t = jnp.dot(s, wa_ref[...], preferred_element_type=jnp.float32)
        t = jnp.maximum(t + ba_ref[...], 0.0)
        u = jnp.dot(t, wb_ref[...], preferred_element_type=jnp.float32)
        h3 = jnp.maximum(u + bb_ref[...], 0.0)
        seg = (bt_ref[...] == lax.broadcasted_iota(jnp.int32, (G, 1), 0)
               ).astype(jnp.float32)
        sums = jnp.dot(seg, h3, preferred_element_type=jnp.float32)
        counts = jnp.sum(seg, axis=1, keepdims=True)
        pooled = sums / jnp.maximum(counts, 1.0)
        out = jnp.dot(pooled, wc_ref[...], preferred_element_type=jnp.float32)
        out_ref[...] = out + bc_ref[...]

    return pl.pallas_call(
        body,
        out_shape=jax.ShapeDtypeStruct((G, C), jnp.float32),
    )(h, agg, Wa, ba.reshape(1, H), Wb, bb.reshape(1, H), batch_t, Wc,
      bc.reshape(1, C))


def kernel(x, edge_index, batch, W1_0, b1_0, W2_0, b2_0, W1_1, b1_1, W2_1,
           b2_1, W1_2, b1_2, W2_2, b2_2, Wc, bc):
    rpw = E // NW
    dpw = EPW - rpw
    dummy_src = (jnp.arange(NW * dpw, dtype=jnp.int32) * 37 % N).reshape(
        NW, dpw)
    dummy_dst = (N + jnp.arange(NW * dpw, dtype=jnp.int32) % (N_PAD - N)
                 ).reshape(NW, dpw)
    src_p = jnp.concatenate(
        [edge_index[0].reshape(NW, rpw), dummy_src], axis=1).reshape(-1)
    dst_p = jnp.concatenate(
        [edge_index[1].reshape(NW, rpw), dummy_dst], axis=1).reshape(-1)
    zrows = jnp.zeros((RPS, D), jnp.float32)
    batch_t = batch.reshape(1, N)

    agg = _sc_aggregate(x, src_p, dst_p, zrows)
    agg = agg.reshape(NC, N_PAD, D)
    h = _tc_layer(x, agg, W1_0, b1_0, W2_0, b2_0)
    agg = _sc_aggregate(h, src_p, dst_p, zrows).reshape(NC, N_PAD, D)
    h = _tc_layer(h, agg, W1_1, b1_1, W2_1, b2_1)
    agg = _sc_aggregate(h, src_p, dst_p, zrows).reshape(NC, N_PAD, D)
    return _tc_final(h, agg, W1_2, b1_2, W2_2, b2_2, batch_t, Wc, bc)


# speedup vs baseline: 1.2212x; 1.0189x over previous
import functools

import jax
import jax.numpy as jnp
from jax import lax
from jax.experimental import pallas as pl
from jax.experimental.pallas import tpu as pltpu
from jax.experimental.pallas import tpu_sc as plsc

N = 10000
E = 320000
D = 128
H = 128
C = 10
G = 64

NC = 2
NS = 16
NW = NC * NS
CHUNK = 80
N_CHUNKS = 128
GRP = 32
N_GRP = N_CHUNKS // GRP
EPW = N_CHUNKS * CHUNK
E_PAD = EPW * NW
N_PAD = 10240
RPS = N_PAD // NS


def _sc_aggregate(h, src_p, dst_p, zrows):
    mesh = plsc.VectorSubcoreMesh(core_axis_name="c", subcore_axis_name="s")

    @functools.partial(
        pl.kernel,
        mesh=mesh,
        out_type=jax.ShapeDtypeStruct((NC * N_PAD, D), jnp.float32),
        scratch_types=[
            pltpu.VMEM((GRP * CHUNK,), jnp.int32),
            pltpu.VMEM((GRP, CHUNK), jnp.int32),
            pltpu.VMEM((CHUNK, D), jnp.float32),
            pltpu.VMEM((CHUNK, D), jnp.float32),
            pltpu.VMEM((CHUNK, D), jnp.float32),
            pltpu.VMEM((CHUNK, D), jnp.float32),
            pltpu.VMEM_SHARED((N_PAD, D), jnp.float32),
            pltpu.SemaphoreType.DMA,
            pltpu.SemaphoreType.DMA,
            pltpu.SemaphoreType.DMA,
            pltpu.SemaphoreType.DMA,
            pltpu.SemaphoreType.DMA,
            pltpu.SemaphoreType.DMA,
            pltpu.SemaphoreType.DMA,
            pltpu.SemaphoreType.DMA,
        ],
    )
    def k(h_hbm, src_hbm, dst4_hbm, z_hbm, out_hbm, sidx, didx, rows0, rows1,
          rows2, rows3, acc, gsem0, gsem1, gsem2, gsem3, ssem0, ssem1, ssem2,
          ssem3):
        rows_b = [rows0, rows1, rows2, rows3]
        gsem_b = [gsem0, gsem1, gsem2, gsem3]
        ssem_b = [ssem0, ssem1, ssem2, ssem3]
        cid = lax.axis_index("c")
        sid = lax.axis_index("s")
        wid = sid * NC + cid

        pltpu.sync_copy(z_hbm, acc.at[pl.ds(sid * RPS, RPS)])
        plsc.subcore_barrier()

        def gather(i, rows, sem):
            return pltpu.async_copy(
                h_hbm.at[sidx.at[pl.ds(i * CHUNK, CHUNK)]], rows, sem)

        def gather_wait(i, rows, sem):
            pltpu.make_async_copy(
                h_hbm.at[sidx.at[pl.ds(i * CHUNK, CHUNK)]], rows, sem).wait()

        @pl.loop(0, N_GRP)
        def _(g):
            c0 = pltpu.async_copy(
                src_hbm.at[pl.ds(wid * EPW + g * GRP * CHUNK, GRP * CHUNK)],
                sidx, gsem0)
            c1 = pltpu.async_copy(dst4_hbm.at[wid, g], didx, gsem1)
            c0.wait()
            c1.wait()

            for b in range(4):
                gather(b, rows_b[b], gsem_b[b])

            @pl.loop(0, GRP - 4, step=4)
            def _(i):
                ss = []
                for b in range(4):
                    gather_wait(i + b, rows_b[b], gsem_b[b])
                    ss.append(pltpu.async_copy(
                        rows_b[b], acc.at[didx.at[i + b]], ssem_b[b],
                        add=True))
                for b in range(4):
                    ss[b].wait()
                    gather(i + b + 4, rows_b[b], gsem_b[b])

            last = GRP - 4
            for b in range(4):
                gather_wait(last + b, rows_b[b], gsem_b[b])
                pltpu.sync_copy(rows_b[b], acc.at[didx.at[last + b]],
                                add=True)

        plsc.subcore_barrier()
        out_row = cid * N_PAD + sid * RPS
        pltpu.sync_copy(acc.at[pl.ds(sid * RPS, RPS)],
                        out_hbm.at[pl.ds(out_row, RPS)])

    return k(h, src_p, dst_p.reshape(NW, N_GRP, GRP, CHUNK), zrows)


def _tc_layer(h, agg, Wa, ba, Wb, bb):
    def body(h_ref, a_ref, wa_ref, ba_ref, wb_ref, bb_ref, out_ref):
        s = h_ref[...] + a_ref[0, :N, :] + a_ref[1, :N, :]
        t = jnp.dot(s, wa_ref[...], preferred_element_type=jnp.float32)
        t = jnp.maximum(t + ba_ref[...], 0.0)
        u = jnp.dot(t, wb_ref[...], preferred_element_type=jnp.float32)
        out_ref[...] = jnp.maximum(u + bb_ref[...], 0.0)

    return pl.pallas_call(
        body,
        out_shape=jax.ShapeDtypeStruct((N, H), jnp.float32),
    )(h, agg, Wa, ba.reshape(1, H), Wb, bb.reshape(1, H))


def _tc_final(h, agg, Wa, ba, Wb, bb, batch_t, Wc, bc):
    def body(h_ref, a_ref, wa_ref, ba_ref, wb_ref, bb_ref, bt_ref, wc_ref,
             bc_ref, out_ref):
        s = h_ref[...] + a_ref[0, :N, :] + a_ref[1, :N, :]
        t = jnp.dot(s, wa_ref[...], preferred_element_type=jnp.float32)
        t = jnp.maximum(t + ba_ref[...], 0.0)
        u = jnp.dot(t, wb_ref[...], preferred_element_type=jnp.float32)
        h3 = jnp.maximum(u + bb_ref[...], 0.0)
        seg = (bt_ref[...] == lax.broadcasted_iota(jnp.int32, (G, 1), 0)
               ).astype(jnp.float32)
        sums = jnp.dot(seg, h3, preferred_element_type=jnp.float32)
        counts = jnp.sum(seg, axis=1, keepdims=True)
        pooled = sums / jnp.maximum(counts, 1.0)
        out = jnp.dot(pooled, wc_ref[...], preferred_element_type=jnp.float32)
        out_ref[...] = out + bc_ref[...]

    return pl.pallas_call(
        body,
        out_shape=jax.ShapeDtypeStruct((G, C), jnp.float32),
    )(h, agg, Wa, ba.reshape(1, H), Wb, bb.reshape(1, H), batch_t, Wc,
      bc.reshape(1, C))


def kernel(x, edge_index, batch, W1_0, b1_0, W2_0, b2_0, W1_1, b1_1, W2_1,
           b2_1, W1_2, b1_2, W2_2, b2_2, Wc, bc):
    rpw = E // NW
    dpw = EPW - rpw
    dummy_src = (jnp.arange(NW * dpw, dtype=jnp.int32) * 37 % N).reshape(
        NW, dpw)
    dummy_dst = (N + jnp.arange(NW * dpw, dtype=jnp.int32) % (N_PAD - N)
                 ).reshape(NW, dpw)
    src_p = jnp.concatenate(
        [edge_index[0].reshape(NW, rpw), dummy_src], axis=1).reshape(-1)
    dst_p = jnp.concatenate(
        [edge_index[1].reshape(NW, rpw), dummy_dst], axis=1).reshape(-1)
    zrows = jnp.zeros((RPS, D), jnp.float32)
    batch_t = batch.reshape(1, N)

    agg = _sc_aggregate(x, src_p, dst_p, zrows)
    agg = agg.reshape(NC, N_PAD, D)
    h = _tc_layer(x, agg, W1_0, b1_0, W2_0, b2_0)
    agg = _sc_aggregate(h, src_p, dst_p, zrows).reshape(NC, N_PAD, D)
    h = _tc_layer(h, agg, W1_1, b1_1, W2_1, b2_1)
    agg = _sc_aggregate(h, src_p, dst_p, zrows).reshape(NC, N_PAD, D)
    return _tc_final(h, agg, W1_2, b1_2, W2_2, b2_2, batch_t, Wc, bc)


# speedup vs baseline: 1.2920x; 1.0579x over previous
import functools

import jax
import jax.numpy as jnp
from jax import lax
from jax.experimental import pallas as pl
from jax.experimental.pallas import tpu as pltpu
from jax.experimental.pallas import tpu_sc as plsc

N = 10000
E = 320000
D = 128
H = 128
C = 10
G = 64

NC = 2
NS = 16
NW = NC * NS
CHUNK = 80
N_CHUNKS = 128
GRP = 32
N_GRP = N_CHUNKS // GRP
EPW = N_CHUNKS * CHUNK
E_PAD = EPW * NW
N_PAD = 10240
RPS = N_PAD // NS


def _sc_aggregate(h, src_p, dst_p, zrows):
    mesh = plsc.VectorSubcoreMesh(core_axis_name="c", subcore_axis_name="s")

    @functools.partial(
        pl.kernel,
        mesh=mesh,
        out_type=jax.ShapeDtypeStruct((NC * N_PAD, D), jnp.float32),
        scratch_types=[
            pltpu.VMEM((GRP * CHUNK,), jnp.int32),
            pltpu.VMEM((GRP, CHUNK), jnp.int32),
            pltpu.VMEM((CHUNK, D), jnp.float32),
            pltpu.VMEM((CHUNK, D), jnp.float32),
            pltpu.VMEM((CHUNK, D), jnp.float32),
            pltpu.VMEM((CHUNK, D), jnp.float32),
            pltpu.VMEM_SHARED((N_PAD, D), jnp.float32),
            pltpu.SemaphoreType.DMA,
            pltpu.SemaphoreType.DMA,
            pltpu.SemaphoreType.DMA,
            pltpu.SemaphoreType.DMA,
            pltpu.SemaphoreType.DMA,
            pltpu.SemaphoreType.DMA,
            pltpu.SemaphoreType.DMA,
            pltpu.SemaphoreType.DMA,
        ],
    )
    def k(h_hbm, src_hbm, dst4_hbm, z_hbm, out_hbm, sidx, didx, rows0, rows1,
          rows2, rows3, acc, gsem0, gsem1, gsem2, gsem3, ssem0, ssem1, ssem2,
          ssem3):
        rows_b = [rows0, rows1, rows2, rows3]
        gsem_b = [gsem0, gsem1, gsem2, gsem3]
        ssem_b = [ssem0, ssem1, ssem2, ssem3]
        cid = lax.axis_index("c")
        sid = lax.axis_index("s")
        wid = sid * NC + cid

        def gather(i, rows, sem):
            return pltpu.async_copy(
                h_hbm.at[sidx.at[pl.ds(i * CHUNK, CHUNK)]], rows, sem)

        def gather_wait(i, rows, sem):
            pltpu.make_async_copy(
                h_hbm.at[sidx.at[pl.ds(i * CHUNK, CHUNK)]], rows, sem).wait()

        def stage_idx(g):
            c0 = pltpu.async_copy(
                src_hbm.at[pl.ds(wid * EPW + g * GRP * CHUNK, GRP * CHUNK)],
                sidx, gsem0)
            c1 = pltpu.async_copy(dst4_hbm.at[wid, g], didx, gsem1)
            return c0, c1

        def run_group():
            for b in range(4):
                gather(b, rows_b[b], gsem_b[b])

            @pl.loop(0, GRP - 4, step=4)
            def _(i):
                ss = []
                for b in range(4):
                    gather_wait(i + b, rows_b[b], gsem_b[b])
                    ss.append(pltpu.async_copy(
                        rows_b[b], acc.at[didx.at[i + b]], ssem_b[b],
                        add=True))
                for b in range(4):
                    ss[b].wait()
                    gather(i + b + 4, rows_b[b], gsem_b[b])

            last = GRP - 4
            for b in range(4):
                gather_wait(last + b, rows_b[b], gsem_b[b])
                pltpu.sync_copy(rows_b[b], acc.at[didx.at[last + b]],
                                add=True)

        z = pltpu.async_copy(z_hbm, acc.at[pl.ds(sid * RPS, RPS)], ssem0)
        c0, c1 = stage_idx(0)
        z.wait()
        plsc.subcore_barrier()
        c0.wait()
        c1.wait()
        run_group()

        @pl.loop(1, N_GRP)
        def _(g):
            c0, c1 = stage_idx(g)
            c0.wait()
            c1.wait()
            run_group()

        plsc.subcore_barrier()
        out_row = cid * N_PAD + sid * RPS
        pltpu.sync_copy(acc.at[pl.ds(sid * RPS, RPS)],
                        out_hbm.at[pl.ds(out_row, RPS)])

    return k(h, src_p, dst_p.reshape(NW, N_GRP, GRP, CHUNK), zrows)


def _tc_layer(h, agg, Wa, ba, Wb, bb):
    def body(h_ref, a_ref, wa_ref, ba_ref, wb_ref, bb_ref, out_ref):
        s = h_ref[...] + a_ref[0, :N, :] + a_ref[1, :N, :]
        t = jnp.dot(s, wa_ref[...], preferred_element_type=jnp.float32)
        t = jnp.maximum(t + ba_ref[...], 0.0)
        u = jnp.dot(t, wb_ref[...], preferred_element_type=jnp.float32)
        out_ref[...] = jnp.maximum(u + bb_ref[...], 0.0)

    return pl.pallas_call(
        body,
        out_shape=jax.ShapeDtypeStruct((N, H), jnp.float32),
    )(h, agg, Wa, ba.reshape(1, H), Wb, bb.reshape(1, H))


def _tc_final(h, agg, Wa, ba, Wb, bb, batch_t, Wc, bc):
    def body(h_ref, a_ref, wa_ref, ba_ref, wb_ref, bb_ref, bt_ref, wc_ref,
             bc_ref, out_ref):
        s = h_ref[...] + a_ref[0, :N, :] + a_ref[1, :N, :]
        t = jnp.dot(s, wa_ref[...], preferred_element_type=jnp.float32)
        t = jnp.maximum(t + ba_ref[...], 0.0)
        u = jnp.dot(t, wb_ref[...], preferred_element_type=jnp.float32)
        h3 = jnp.maximum(u + bb_ref[...], 0.0)
        seg = (bt_ref[...] == lax.broadcasted_iota(jnp.int32, (G, 1), 0)
               ).astype(jnp.float32)
        sums = jnp.dot(seg, h3, preferred_element_type=jnp.float32)
        counts = jnp.sum(seg, axis=1, keepdims=True)
        pooled = sums / jnp.maximum(counts, 1.0)
        out = jnp.dot(pooled, wc_ref[...], preferred_element_type=jnp.float32)
        out_ref[...] = out + bc_ref[...]

    return pl.pallas_call(
        body,
        out_shape=jax.ShapeDtypeStruct((G, C), jnp.float32),
    )(h, agg, Wa, ba.reshape(1, H), Wb, bb.reshape(1, H), batch_t, Wc,
      bc.reshape(1, C))


def kernel(x, edge_index, batch, W1_0, b1_0, W2_0, b2_0, W1_1, b1_1, W2_1,
           b2_1, W1_2, b1_2, W2_2, b2_2, Wc, bc):
    rpw = E // NW
    dpw = EPW - rpw
    dummy_src = (jnp.arange(NW * dpw, dtype=jnp.int32) * 37 % N).reshape(
        NW, dpw)
    dummy_dst = (N + jnp.arange(NW * dpw, dtype=jnp.int32) % (N_PAD - N)
                 ).reshape(NW, dpw)
    src_p = jnp.concatenate(
        [edge_index[0].reshape(NW, rpw), dummy_src], axis=1).reshape(-1)
    dst_p = jnp.concatenate(
        [edge_index[1].reshape(NW, rpw), dummy_dst], axis=1).reshape(-1)
    zrows = jnp.zeros((RPS, D), jnp.float32)
    batch_t = batch.reshape(1, N)

    agg = _sc_aggregate(x, src_p, dst_p, zrows)
    agg = agg.reshape(NC, N_PAD, D)
    h = _tc_layer(x, agg, W1_0, b1_0, W2_0, b2_0)
    agg = _sc_aggregate(h, src_p, dst_p, zrows).reshape(NC, N_PAD, D)
    h = _tc_layer(h, agg, W1_1, b1_1, W2_1, b2_1)
    agg = _sc_aggregate(h, src_p, dst_p, zrows).reshape(NC, N_PAD, D)
    return _tc_final(h, agg, W1_2, b1_2, W2_2, b2_2, batch_t, Wc, bc)
